# Initial kernel scaffold; baseline (speedup 1.0000x reference)
import jax
import jax.numpy as jnp
from jax.experimental import pallas as pl


def kernel(e1, rel, edge_index, emb_e, emb_rel, W_msg, W_ih, W_hh, b_ih, b_hh):
    raise NotImplementedError("write your pallas kernel here")


# speedup vs baseline: 3.1649x; 3.1649x over previous
import functools

import jax
import jax.numpy as jnp
from jax import lax
from jax.experimental import pallas as pl
from jax.experimental.pallas import tpu as pltpu
from jax.experimental.pallas import tpu_sc as plsc

NC = 2
NS = 16
NW = NC * NS
EG = 128
GRP = 128
NCHUNK = 2

_DOT = functools.partial(jnp.dot, preferred_element_type=jnp.float32,
                         precision=jax.lax.Precision.HIGHEST)


def _sc_mesh():
    return plsc.VectorSubcoreMesh(core_axis_name="c", subcore_axis_name="s",
                                  num_cores=NC, num_subcores=NS)


def _sc_segment_sum(table, eidx, n_ent, n_grp, acc_rows):
    D = table.shape[1]
    zrows = acc_rows // (NS * EG)
    ch = n_grp // NCHUNK

    @functools.partial(
        pl.kernel,
        mesh=_sc_mesh(),
        out_type=(jax.ShapeDtypeStruct((n_ent, D), jnp.float32),
                  jax.ShapeDtypeStruct((n_ent, D), jnp.float32)),
        scratch_types=[
            pltpu.VMEM((ch, EG), jnp.int32),
            pltpu.VMEM((ch, EG), jnp.int32),
            pltpu.VMEM((EG, D), jnp.float32),
            pltpu.VMEM((EG, D), jnp.float32),
            pltpu.VMEM_SHARED((acc_rows, D), jnp.float32),
            pltpu.SemaphoreType.DMA,
            pltpu.SemaphoreType.DMA,
        ],
    )
    def k(table_hbm, eidx_hbm, out0, out1, src_v, dst_v, rb0, rb1, acc,
          sem0, sem1):
        c = lax.axis_index("c")
        s = lax.axis_index("s")
        wid = s * NC + c

        def zrow(i, carry):
            for j in range(D // 16):
                rb0[i, pl.ds(j * 16, 16)] = jnp.zeros((16,), jnp.float32)
            return carry
        lax.fori_loop(0, EG, zrow, 0)
        for z in range(zrows):
            pltpu.sync_copy(rb0, acc.at[pl.ds((s * zrows + z) * EG, EG)])

        plsc.subcore_barrier()

        def start_gather(g, rb, sem):
            pltpu.async_copy(table_hbm.at[src_v.at[g]], rb, sem)

        def wait_gather(rb, sem):
            pltpu.make_async_copy(table_hbm.at[src_v.at[0]], rb, sem).wait()

        for chunk in range(NCHUNK):
            pltpu.sync_copy(eidx_hbm.at[0, wid, pl.ds(chunk * ch, ch)], src_v)
            pltpu.sync_copy(eidx_hbm.at[1, wid, pl.ds(chunk * ch, ch)], dst_v)
            start_gather(0, rb0, sem0)

            def body(i, carry):
                g0 = 2 * i
                start_gather(g0 + 1, rb1, sem1)
                wait_gather(rb0, sem0)
                pltpu.sync_copy(rb0, acc.at[dst_v.at[g0]], add=True)
                g2 = jnp.where(g0 + 2 >= ch, 0, g0 + 2)
                start_gather(g2, rb0, sem0)
                wait_gather(rb1, sem1)
                pltpu.sync_copy(rb1, acc.at[dst_v.at[g0 + 1]], add=True)
                return carry
            lax.fori_loop(0, ch // 2, body, 0)
            wait_gather(rb0, sem0)

        plsc.subcore_barrier()

        rp = (n_ent // NS) & ~7
        rem = n_ent - NS * rp
        @pl.when(c == 0)
        def _():
            pltpu.sync_copy(acc.at[pl.ds(s * rp, rp)],
                            out0.at[pl.ds(s * rp, rp)])
        @pl.when(c == 1)
        def _():
            pltpu.sync_copy(acc.at[pl.ds(s * rp, rp)],
                            out1.at[pl.ds(s * rp, rp)])
        if rem:
            @pl.when((c == 0) & (s == NS - 1))
            def _():
                pltpu.sync_copy(acc.at[pl.ds(NS * rp, rem)],
                                out0.at[pl.ds(NS * rp, rem)])
            @pl.when((c == 1) & (s == NS - 1))
            def _():
                pltpu.sync_copy(acc.at[pl.ds(NS * rp, rem)],
                                out1.at[pl.ds(NS * rp, rem)])

    return k(table, eidx)


def _sc_select(h1, q0, q1, emb_rel, e1g, relg, n_sel_grp):
    D = h1.shape[1]
    B = n_sel_grp * GRP

    @functools.partial(
        pl.kernel,
        mesh=_sc_mesh(),
        out_type=(jax.ShapeDtypeStruct((B, D), jnp.float32),
                  jax.ShapeDtypeStruct((B, D), jnp.float32),
                  jax.ShapeDtypeStruct((B, D), jnp.float32),
                  jax.ShapeDtypeStruct((B, D), jnp.float32)),
        scratch_types=[
            pltpu.VMEM((1, GRP), jnp.int32),
            pltpu.VMEM((GRP, D), jnp.float32),
            pltpu.SemaphoreType.DMA,
        ],
    )
    def k(h1_hbm, q0_hbm, q1_hbm, rel_hbm, e1_hbm, relg_hbm,
          oh1, oq0, oq1, orel, idx_v, rows, sem):
        c = lax.axis_index("c")
        s = lax.axis_index("s")
        wid = s * NC + c
        tbl = wid // n_sel_grp
        g = wid % n_sel_grp

        @pl.when(tbl == 3)
        def _():
            pltpu.sync_copy(relg_hbm.at[pl.ds(g, 1)], idx_v)
        @pl.when(tbl < 3)
        def _():
            pltpu.sync_copy(e1_hbm.at[pl.ds(g, 1)], idx_v)

        @pl.when(tbl == 0)
        def _():
            pltpu.async_copy(h1_hbm.at[idx_v.at[0]], rows, sem).wait()
            pltpu.sync_copy(rows, oh1.at[pl.ds(g * GRP, GRP)])
        @pl.when(tbl == 1)
        def _():
            pltpu.async_copy(q0_hbm.at[idx_v.at[0]], rows, sem).wait()
            pltpu.sync_copy(rows, oq0.at[pl.ds(g * GRP, GRP)])
        @pl.when(tbl == 2)
        def _():
            pltpu.async_copy(q1_hbm.at[idx_v.at[0]], rows, sem).wait()
            pltpu.sync_copy(rows, oq1.at[pl.ds(g * GRP, GRP)])
        @pl.when(tbl == 3)
        def _():
            pltpu.async_copy(rel_hbm.at[idx_v.at[0]], rows, sem).wait()
            pltpu.sync_copy(rows, orel.at[pl.ds(g * GRP, GRP)])

    return k(h1, q0, q1, emb_rel, e1g, relg)


def _tc_prep(h, W_msg, W_hh, b_hh, rb):
    V, D = h.shape
    D3 = W_hh.shape[1]

    def body(h_ref, wm_ref, wh_ref, bh_ref, hw_ref, gh_ref):
        hb = h_ref[...]
        hw_ref[...] = _DOT(hb, wm_ref[...])
        gh_ref[...] = _DOT(hb, wh_ref[...]) + bh_ref[...]

    return pl.pallas_call(
        body,
        grid=(V // rb,),
        in_specs=[
            pl.BlockSpec((rb, D), lambda i: (i, 0)),
            pl.BlockSpec((D, D), lambda i: (0, 0)),
            pl.BlockSpec((D, D3), lambda i: (0, 0)),
            pl.BlockSpec((1, D3), lambda i: (0, 0)),
        ],
        out_specs=[
            pl.BlockSpec((rb, D), lambda i: (i, 0)),
            pl.BlockSpec((rb, D3), lambda i: (i, 0)),
        ],
        out_shape=[
            jax.ShapeDtypeStruct((V, D), jnp.float32),
            jax.ShapeDtypeStruct((V, D3), jnp.float32),
        ],
    )(h, W_msg, W_hh, b_hh)


def _gru_combine(a, gh, h, W_ih, b_ih):
    gi = _DOT(a, W_ih) + b_ih
    D = h.shape[-1]
    i_r, i_z, i_n = gi[:, :D], gi[:, D:2 * D], gi[:, 2 * D:]
    h_r, h_z, h_n = gh[:, :D], gh[:, D:2 * D], gh[:, 2 * D:]
    r = jax.nn.sigmoid(i_r + h_r)
    z = jax.nn.sigmoid(i_z + h_z)
    n = jnp.tanh(i_n + r * h_n)
    return (1.0 - z) * n + z * h


def _tc_gru_prep(p0, p1, gh, h0, W_ih, b_ih, W_msg, rb):
    V, D = h0.shape
    D3 = W_ih.shape[1]

    def body(p0_ref, p1_ref, gh_ref, h0_ref, wih_ref, bih_ref, wm_ref,
             h1_ref, hw2_ref):
        a = p0_ref[...] + p1_ref[...]
        h1 = _gru_combine(a, gh_ref[...], h0_ref[...], wih_ref[...],
                          bih_ref[...])
        h1_ref[...] = h1
        hw2_ref[...] = _DOT(h1, wm_ref[...])

    return pl.pallas_call(
        body,
        grid=(V // rb,),
        in_specs=[
            pl.BlockSpec((rb, D), lambda i: (i, 0)),
            pl.BlockSpec((rb, D), lambda i: (i, 0)),
            pl.BlockSpec((rb, D3), lambda i: (i, 0)),
            pl.BlockSpec((rb, D), lambda i: (i, 0)),
            pl.BlockSpec((D, D3), lambda i: (0, 0)),
            pl.BlockSpec((1, D3), lambda i: (0, 0)),
            pl.BlockSpec((D, D), lambda i: (0, 0)),
        ],
        out_specs=[
            pl.BlockSpec((rb, D), lambda i: (i, 0)),
            pl.BlockSpec((rb, D), lambda i: (i, 0)),
        ],
        out_shape=[
            jax.ShapeDtypeStruct((V, D), jnp.float32),
            jax.ShapeDtypeStruct((V, D), jnp.float32),
        ],
    )(p0, p1, gh, h0, W_ih, b_ih, W_msg)


def _tc_final(h1s, q0s, q1s, rels, W_ih, b_ih, W_hh, b_hh, emb_e, cb):
    B, D = h1s.shape
    D3 = W_ih.shape[1]
    V = emb_e.shape[0]

    def body(h1_ref, q0_ref, q1_ref, rel_ref, wih_ref, bih_ref, whh_ref,
             bhh_ref, embe_ref, out_ref, q_ref):
        @pl.when(pl.program_id(0) == 0)
        def _():
            a = q0_ref[...] + q1_ref[...]
            h2 = _gru_combine(a, _DOT(h1_ref[...], whh_ref[...]) +
                              bhh_ref[...], h1_ref[...], wih_ref[...],
                              bih_ref[...])
            q_ref[...] = h2 * rel_ref[...]
        score = lax.dot_general(
            q_ref[...], embe_ref[...], (((1,), (1,)), ((), ())),
            preferred_element_type=jnp.float32,
            precision=jax.lax.Precision.HIGHEST)
        out_ref[...] = jax.nn.sigmoid(score)

    return pl.pallas_call(
        body,
        grid=(-(-V // cb),),
        in_specs=[
            pl.BlockSpec((B, D), lambda i: (0, 0)),
            pl.BlockSpec((B, D), lambda i: (0, 0)),
            pl.BlockSpec((B, D), lambda i: (0, 0)),
            pl.BlockSpec((B, D), lambda i: (0, 0)),
            pl.BlockSpec((D, D3), lambda i: (0, 0)),
            pl.BlockSpec((1, D3), lambda i: (0, 0)),
            pl.BlockSpec((D, D3), lambda i: (0, 0)),
            pl.BlockSpec((1, D3), lambda i: (0, 0)),
            pl.BlockSpec((cb, D), lambda i: (i, 0)),
        ],
        out_specs=pl.BlockSpec((B, cb), lambda i: (0, i)),
        out_shape=jax.ShapeDtypeStruct((B, V), jnp.float32),
        scratch_shapes=[pltpu.VMEM((B, D), jnp.float32)],
    )(h1s, q0s, q1s, rels, W_ih, b_ih, W_hh, b_hh, emb_e)


def kernel(e1, rel, edge_index, emb_e, emb_rel, W_msg, W_ih, W_hh, b_ih, b_hh):
    V, D = emb_e.shape
    E = edge_index.shape[1]
    B = e1.shape[0]
    D3 = W_ih.shape[1]

    n_grp = -(-E // (NW * EG))
    gq = 2 * NCHUNK
    n_grp = -(-n_grp // gq) * gq
    e_pad = NW * n_grp * EG - E
    src = jnp.concatenate([edge_index[0].astype(jnp.int32),
                           jnp.zeros((e_pad,), jnp.int32)])
    dst = jnp.concatenate([edge_index[1].astype(jnp.int32),
                           jnp.full((e_pad,), V, jnp.int32)])
    eidx = jnp.stack([src, dst]).reshape(2, NW, n_grp, EG)

    acc_rows = -(-(V + 1) // (NS * EG)) * (NS * EG)
    n_sel_grp = B // GRP
    e1g = e1[:, 0].astype(jnp.int32).reshape(n_sel_grp, GRP)
    relg = rel[:, 0].astype(jnp.int32).reshape(n_sel_grp, GRP)
    b_ih2 = b_ih.reshape(1, D3)
    b_hh2 = b_hh.reshape(1, D3)
    rb = 1000 if V % 1000 == 0 else 8 * (V // 8)

    hw1, gh1 = _tc_prep(emb_e, W_msg, W_hh, b_hh2, rb)
    p0, p1 = _sc_segment_sum(hw1, eidx, V, n_grp, acc_rows)
    h1, hw2 = _tc_gru_prep(p0, p1, gh1, emb_e, W_ih, b_ih2, W_msg, rb)

    q0, q1 = _sc_segment_sum(hw2, eidx, V, n_grp, acc_rows)
    h1s, q0s, q1s, rels = _sc_select(h1, q0, q1, emb_rel, e1g, relg,
                                     n_sel_grp)

    return _tc_final(h1s, q0s, q1s, rels, W_ih, b_ih2, W_hh, b_hh2, emb_e,
                     1024)
